# Initial kernel scaffold; baseline (speedup 1.0000x reference)
import jax
import jax.numpy as jnp
from jax.experimental import pallas as pl


def kernel(items, query_words, word_emb, q_weight, q_bias):
    raise NotImplementedError("write your pallas kernel here")


# speedup vs baseline: 17.3260x; 17.3260x over previous
import functools

import jax
import jax.numpy as jnp
from jax import lax
from jax.experimental import pallas as pl
from jax.experimental.pallas import tpu as pltpu
from jax.experimental.pallas import tpu_sc as plsc

B = 16384
H = 200
E = 64
NW = 32
RW = B // NW
C = 4
NCHUNK = RW // C
NG = 2 * C


def _sc_body(table_hbm, qw_hbm, out_hbm, idx_v, emb_v, out_v, sem):
    wid = lax.axis_index("s") * 2 + lax.axis_index("c")
    base = wid * RW

    def chunk_body(ci, carry):
        cbase = base + ci * C
        pltpu.sync_copy(qw_hbm.at[pl.ds(cbase * 2, NG)], idx_v)
        descs = []
        for j in range(NG):
            descs.append(
                pltpu.async_copy(table_hbm.at[idx_v.at[j]], emb_v.at[j], sem))
        for d in descs:
            d.wait()
        for r in range(C):
            def h_body(h, accs):
                a0, a1, a2, a3 = accs
                for j in (2 * r, 2 * r + 1):
                    a0 = a0 + emb_v[j, h, pl.ds(0, 16)]
                    a1 = a1 + emb_v[j, h, pl.ds(16, 16)]
                    a2 = a2 + emb_v[j, h, pl.ds(32, 16)]
                    a3 = a3 + emb_v[j, h, pl.ds(48, 16)]
                return (a0, a1, a2, a3)

            z = jnp.zeros((16,), jnp.float32)
            a0, a1, a2, a3 = lax.fori_loop(0, 100, h_body, (z, z, z, z))
            out_v[r, pl.ds(0, 16)] = a0
            out_v[r, pl.ds(16, 16)] = a1
            out_v[r, pl.ds(32, 16)] = a2
            out_v[r, pl.ds(48, 16)] = a3
        pltpu.sync_copy(out_v, out_hbm.at[pl.ds(cbase, C)])
        return carry

    lax.fori_loop(0, NCHUNK, chunk_body, 0)


@jax.jit
def _sc_sums(word_emb, qw2):
    mesh = plsc.VectorSubcoreMesh(core_axis_name="c", subcore_axis_name="s")
    f = pl.kernel(
        _sc_body,
        mesh=mesh,
        compiler_params=pltpu.CompilerParams(use_tc_tiling_on_sc=False),
        out_type=jax.ShapeDtypeStruct((B, E), jnp.float32),
        scratch_types=[
            pltpu.VMEM((NG, 100), jnp.int32),
            pltpu.VMEM((NG, 100, E), jnp.float32),
            pltpu.VMEM((C, E), jnp.float32),
            pltpu.SemaphoreType.DMA,
        ],
    )
    return f(word_emb, qw2)


def _proj_body(x_ref, w_ref, b_ref, o_ref):
    x = x_ref[...]
    o_ref[...] = jnp.tanh(
        jnp.dot(x, w_ref[...], preferred_element_type=jnp.float32) + b_ref[...])


@jax.jit
def _proj(sums, wt, bias):
    blk = 2048
    return pl.pallas_call(
        _proj_body,
        grid=(B // blk,),
        in_specs=[
            pl.BlockSpec((blk, E), lambda i: (i, 0)),
            pl.BlockSpec((E, E), lambda i: (0, 0)),
            pl.BlockSpec((1, E), lambda i: (0, 0)),
        ],
        out_specs=pl.BlockSpec((blk, E), lambda i: (i, 0)),
        out_shape=jax.ShapeDtypeStruct((B, E), jnp.float32),
    )(sums, wt, bias)


def kernel(items, query_words, word_emb, q_weight, q_bias):
    qw2 = query_words.reshape(2 * B, 100)
    sums = _sc_sums(word_emb, qw2)
    wt = q_weight.T * (1.0 / H)
    return _proj(sums, wt, q_bias.reshape(1, E))


# speedup vs baseline: 27.3323x; 1.5775x over previous
import functools

import jax
import jax.numpy as jnp
from jax import lax
from jax.experimental import pallas as pl
from jax.experimental.pallas import tpu as pltpu
from jax.experimental.pallas import tpu_sc as plsc

B = 16384
H = 200
E = 64
NW = 32
RW = B // NW
C = 4
NCHUNK = RW // C
NG = 2 * C


def _sc_body(table_hbm, qw_hbm, out_hbm,
             idx0, idx1, emb0, emb1, out_v, sem0, sem1):
    wid = lax.axis_index("s") * 2 + lax.axis_index("c")
    base = wid * RW
    idx_b = (idx0, idx1)
    emb_b = (emb0, emb1)
    sem_b = (sem0, sem1)

    def fire(ci, b):
        pltpu.sync_copy(qw_hbm.at[pl.ds((base + ci * C) * 2, NG)], idx_b[b])
        for j in range(NG):
            pltpu.async_copy(table_hbm.at[idx_b[b].at[j]],
                             emb_b[b].at[j], sem_b[b])

    def drain(b):
        for j in range(NG):
            pltpu.make_async_copy(table_hbm.at[idx_b[b].at[j]],
                                  emb_b[b].at[j], sem_b[b]).wait()

    def accum(ci, b):
        emb_v = emb_b[b]
        for r in range(C):
            def h_body(h, accs):
                a0, a1, a2, a3 = accs
                for j in (2 * r, 2 * r + 1):
                    a0 = a0 + emb_v[j, h, pl.ds(0, 16)]
                    a1 = a1 + emb_v[j, h, pl.ds(16, 16)]
                    a2 = a2 + emb_v[j, h, pl.ds(32, 16)]
                    a3 = a3 + emb_v[j, h, pl.ds(48, 16)]
                return (a0, a1, a2, a3)

            z = jnp.zeros((16,), jnp.float32)
            a0, a1, a2, a3 = lax.fori_loop(0, 100, h_body, (z, z, z, z),
                                           unroll=4)
            out_v[r, pl.ds(0, 16)] = a0
            out_v[r, pl.ds(16, 16)] = a1
            out_v[r, pl.ds(32, 16)] = a2
            out_v[r, pl.ds(48, 16)] = a3
        pltpu.sync_copy(out_v, out_hbm.at[pl.ds(base + ci * C, C)])

    fire(0, 0)
    fire(1, 1)

    def pair_body(cp, carry):
        ci = cp * 2
        drain(0)
        accum(ci, 0)
        fire(ci + 2, 0)
        drain(1)
        accum(ci + 1, 1)
        fire(ci + 3, 1)
        return carry

    lax.fori_loop(0, NCHUNK // 2 - 1, pair_body, 0)
    drain(0)
    accum(NCHUNK - 2, 0)
    drain(1)
    accum(NCHUNK - 1, 1)


@jax.jit
def _sc_sums(word_emb, qw2):
    mesh = plsc.VectorSubcoreMesh(core_axis_name="c", subcore_axis_name="s")
    f = pl.kernel(
        _sc_body,
        mesh=mesh,
        compiler_params=pltpu.CompilerParams(use_tc_tiling_on_sc=False),
        out_type=jax.ShapeDtypeStruct((B, E), jnp.float32),
        scratch_types=[
            pltpu.VMEM((NG, 100), jnp.int32),
            pltpu.VMEM((NG, 100), jnp.int32),
            pltpu.VMEM((NG, 100, E), jnp.float32),
            pltpu.VMEM((NG, 100, E), jnp.float32),
            pltpu.VMEM((C, E), jnp.float32),
            pltpu.SemaphoreType.DMA,
            pltpu.SemaphoreType.DMA,
        ],
    )
    return f(word_emb, qw2)


def _proj_body(x_ref, w_ref, b_ref, o_ref):
    x = x_ref[...]
    o_ref[...] = jnp.tanh(
        jnp.dot(x, w_ref[...], preferred_element_type=jnp.float32) + b_ref[...])


@jax.jit
def _proj(sums, wt, bias):
    blk = 2048
    return pl.pallas_call(
        _proj_body,
        grid=(B // blk,),
        in_specs=[
            pl.BlockSpec((blk, E), lambda i: (i, 0)),
            pl.BlockSpec((E, E), lambda i: (0, 0)),
            pl.BlockSpec((1, E), lambda i: (0, 0)),
        ],
        out_specs=pl.BlockSpec((blk, E), lambda i: (i, 0)),
        out_shape=jax.ShapeDtypeStruct((B, E), jnp.float32),
    )(sums, wt, bias)


def kernel(items, query_words, word_emb, q_weight, q_bias):
    qw2 = query_words.reshape(2 * B, 100)
    sums = _sc_sums(word_emb, qw2)
    wt = q_weight.T * (1.0 / H)
    return _proj(sums, wt, q_bias.reshape(1, E))


# speedup vs baseline: 29.1620x; 1.0669x over previous
import functools

import jax
import jax.numpy as jnp
from jax import lax
from jax.experimental import pallas as pl
from jax.experimental.pallas import tpu as pltpu
from jax.experimental.pallas import tpu_sc as plsc

B = 16384
H = 200
E = 64
NW = 32
RW = B // NW
C = 4
NCHUNK = RW // C
NG = 2 * C


def _sc_body(table_hbm, qw_hbm, out_hbm,
             idx0, idx1, emb0, emb1, out_v, sem0, sem1):
    wid = lax.axis_index("s") * 2 + lax.axis_index("c")
    base = wid * RW
    idx_b = (idx0, idx1)
    emb_b = (emb0, emb1)
    sem_b = (sem0, sem1)

    SPLITS = ((0, 128), (128, 72))

    def fire(ci, b):
        pltpu.sync_copy(qw_hbm.at[pl.ds(base + ci * C, C)], idx_b[b])
        for r in range(C):
            for off, n in SPLITS:
                pltpu.async_copy(
                    table_hbm.at[idx_b[b].at[r, pl.ds(off, n)]],
                    emb_b[b].at[r, pl.ds(off, n)], sem_b[b])

    def drain(b):
        for r in range(C):
            for off, n in SPLITS:
                pltpu.make_async_copy(
                    table_hbm.at[idx_b[b].at[r, pl.ds(off, n)]],
                    emb_b[b].at[r, pl.ds(off, n)], sem_b[b]).wait()

    def accum(ci, b):
        emb_v = emb_b[b]
        for r in range(C):
            def h_body(h, accs):
                a0, a1, a2, a3 = accs
                a0 = a0 + emb_v[r, h, pl.ds(0, 16)]
                a1 = a1 + emb_v[r, h, pl.ds(16, 16)]
                a2 = a2 + emb_v[r, h, pl.ds(32, 16)]
                a3 = a3 + emb_v[r, h, pl.ds(48, 16)]
                return (a0, a1, a2, a3)

            z = jnp.zeros((16,), jnp.float32)
            a0, a1, a2, a3 = lax.fori_loop(0, H, h_body, (z, z, z, z),
                                           unroll=8)
            out_v[r, pl.ds(0, 16)] = a0
            out_v[r, pl.ds(16, 16)] = a1
            out_v[r, pl.ds(32, 16)] = a2
            out_v[r, pl.ds(48, 16)] = a3
        pltpu.sync_copy(out_v, out_hbm.at[pl.ds(base + ci * C, C)])

    fire(0, 0)
    fire(1, 1)

    def pair_body(cp, carry):
        ci = cp * 2
        drain(0)
        accum(ci, 0)
        fire(ci + 2, 0)
        drain(1)
        accum(ci + 1, 1)
        fire(ci + 3, 1)
        return carry

    lax.fori_loop(0, NCHUNK // 2 - 1, pair_body, 0)
    drain(0)
    accum(NCHUNK - 2, 0)
    drain(1)
    accum(NCHUNK - 1, 1)


@jax.jit
def _sc_sums(word_emb, qw2):
    mesh = plsc.VectorSubcoreMesh(core_axis_name="c", subcore_axis_name="s")
    f = pl.kernel(
        _sc_body,
        mesh=mesh,
        compiler_params=pltpu.CompilerParams(use_tc_tiling_on_sc=False),
        out_type=jax.ShapeDtypeStruct((B, E), jnp.float32),
        scratch_types=[
            pltpu.VMEM((C, H), jnp.int32),
            pltpu.VMEM((C, H), jnp.int32),
            pltpu.VMEM((C, H, E), jnp.float32),
            pltpu.VMEM((C, H, E), jnp.float32),
            pltpu.VMEM((C, E), jnp.float32),
            pltpu.SemaphoreType.DMA,
            pltpu.SemaphoreType.DMA,
        ],
    )
    return f(word_emb, qw2)


def _proj_body(x_ref, w_ref, b_ref, o_ref):
    x = x_ref[...]
    o_ref[...] = jnp.tanh(
        jnp.dot(x, w_ref[...], preferred_element_type=jnp.float32) + b_ref[...])


@jax.jit
def _proj(sums, wt, bias):
    blk = 2048
    return pl.pallas_call(
        _proj_body,
        grid=(B // blk,),
        in_specs=[
            pl.BlockSpec((blk, E), lambda i: (i, 0)),
            pl.BlockSpec((E, E), lambda i: (0, 0)),
            pl.BlockSpec((1, E), lambda i: (0, 0)),
        ],
        out_specs=pl.BlockSpec((blk, E), lambda i: (i, 0)),
        out_shape=jax.ShapeDtypeStruct((B, E), jnp.float32),
    )(sums, wt, bias)


def kernel(items, query_words, word_emb, q_weight, q_bias):
    sums = _sc_sums(word_emb, query_words)
    wt = q_weight.T * (1.0 / H)
    return _proj(sums, wt, q_bias.reshape(1, E))


# speedup vs baseline: 37.2327x; 1.2768x over previous
import jax
import jax.numpy as jnp
from jax import lax
from jax.experimental import pallas as pl
from jax.experimental.pallas import tpu as pltpu
from jax.experimental.pallas import tpu_sc as plsc

B = 16384
H = 200
E = 64
NW = 32
RW = B // NW
C = 8
NCHUNK = RW // C
SPLITS = ((0, 128), (128, 72))


def _sc_body(table_hbm, qw_hbm, out_hbm,
             idx0, idx1, emb0, emb1, out_v, sem0, sem1):
    wid = lax.axis_index("s") * 2 + lax.axis_index("c")
    base = wid * RW
    idx_b = (idx0, idx1)
    emb_b = (emb0, emb1)
    sem_b = (sem0, sem1)

    def fire(ci, b):
        pltpu.sync_copy(qw_hbm.at[pl.ds(base + ci * C, C)], idx_b[b])
        for r in range(C):
            for off, n in SPLITS:
                pltpu.async_copy(
                    table_hbm.at[idx_b[b].at[r, pl.ds(off, n)]],
                    emb_b[b].at[r, pl.ds(off, n)], sem_b[b])

    def drain(b):
        for r in range(C):
            for off, n in SPLITS:
                pltpu.make_async_copy(
                    table_hbm.at[idx_b[b].at[r, pl.ds(off, n)]],
                    emb_b[b].at[r, pl.ds(off, n)], sem_b[b]).wait()

    def accum(ci, b):
        emb_v = emb_b[b]
        for r in range(C):
            def h_body(h, accs):
                e0, o0, e1, o1 = accs
                a, bb = plsc.unpack(emb_v[r, h, pl.ds(0, 32)],
                                    format=plsc.PackFormat.INTERLEAVED)
                e0 = e0 + a
                o0 = o0 + bb
                a, bb = plsc.unpack(emb_v[r, h, pl.ds(32, 32)],
                                    format=plsc.PackFormat.INTERLEAVED)
                e1 = e1 + a
                o1 = o1 + bb
                return (e0, o0, e1, o1)

            z = jnp.zeros((16,), jnp.float32)
            e0, o0, e1, o1 = lax.fori_loop(0, H, h_body, (z, z, z, z),
                                           unroll=8)
            out_v[r, pl.ds(0, 32)] = plsc.pack(
                e0, o0, format=plsc.PackFormat.INTERLEAVED)
            out_v[r, pl.ds(32, 32)] = plsc.pack(
                e1, o1, format=plsc.PackFormat.INTERLEAVED)
        pltpu.sync_copy(out_v, out_hbm.at[pl.ds(base + ci * C, C)])

    fire(0, 0)
    fire(1, 1)

    def pair_body(cp, carry):
        ci = cp * 2
        drain(0)
        accum(ci, 0)
        fire(ci + 2, 0)
        drain(1)
        accum(ci + 1, 1)
        fire(ci + 3, 1)
        return carry

    lax.fori_loop(0, NCHUNK // 2 - 1, pair_body, 0)
    drain(0)
    accum(NCHUNK - 2, 0)
    drain(1)
    accum(NCHUNK - 1, 1)


@jax.jit
def _sc_sums(table_bf16, qw):
    mesh = plsc.VectorSubcoreMesh(core_axis_name="c", subcore_axis_name="s")
    f = pl.kernel(
        _sc_body,
        mesh=mesh,
        compiler_params=pltpu.CompilerParams(use_tc_tiling_on_sc=False, needs_layout_passes=False),
        out_type=jax.ShapeDtypeStruct((B, E), jnp.bfloat16),
        scratch_types=[
            pltpu.VMEM((C, H), jnp.int32),
            pltpu.VMEM((C, H), jnp.int32),
            pltpu.VMEM((C, H, E), jnp.bfloat16),
            pltpu.VMEM((C, H, E), jnp.bfloat16),
            pltpu.VMEM((C, E), jnp.bfloat16),
            pltpu.SemaphoreType.DMA,
            pltpu.SemaphoreType.DMA,
        ],
    )
    return f(table_bf16, qw)


def _proj_body(x_ref, w_ref, b_ref, o_ref):
    x = x_ref[...].astype(jnp.float32)
    o_ref[...] = jnp.tanh(
        jnp.dot(x, w_ref[...], preferred_element_type=jnp.float32) + b_ref[...])


@jax.jit
def _proj(sums, wt, bias):
    blk = 2048
    return pl.pallas_call(
        _proj_body,
        grid=(B // blk,),
        in_specs=[
            pl.BlockSpec((blk, E), lambda i: (i, 0)),
            pl.BlockSpec((E, E), lambda i: (0, 0)),
            pl.BlockSpec((1, E), lambda i: (0, 0)),
        ],
        out_specs=pl.BlockSpec((blk, E), lambda i: (i, 0)),
        out_shape=jax.ShapeDtypeStruct((B, E), jnp.float32),
    )(sums, wt, bias)


def kernel(items, query_words, word_emb, q_weight, q_bias):
    sums = _sc_sums(word_emb.astype(jnp.bfloat16), query_words)
    wt = q_weight.T * (1.0 / H)
    return _proj(sums, wt, q_bias.reshape(1, E))


# speedup vs baseline: 41.0215x; 1.1018x over previous
import numpy as np

import jax
import jax.numpy as jnp
from jax import lax
from jax.experimental import pallas as pl
from jax.experimental.pallas import tpu as pltpu
from jax.experimental.pallas import tpu_sc as plsc

B = 16384
H = 200
E = 64
NW = 32
RW = B // NW
C = 16
NCHUNK = RW // C
NIDX = C * H // 128
HC = C // 2
FLAT = HC * H

_G0 = [(k, 0, 128, 128 * k) for k in range(12)] + [(12, 0, 64, 1536)]
_G1 = [(12, 64, 64, 0)] + [(13 + k, 0, 128, 64 + 128 * k) for k in range(12)]
_GATHERS = (_G0, _G1)


def _sc_body(table_hbm, qw_hbm, out_hbm,
             idx0, idx1, embA, embB, out_v, semA, semB):
    wid = lax.axis_index("s") * 2 + lax.axis_index("c")
    base = wid * RW
    idx_b = (idx0, idx1)
    emb_b = (embA, embB)
    sem_b = (semA, semB)

    def stage_idx(ci, p):
        pltpu.sync_copy(qw_hbm.at[pl.ds((base + ci * C) * H // 128, NIDX)],
                        idx_b[p])

    def fire(half, p, eb):
        for row, coff, n, doff in _GATHERS[half]:
            src = (idx_b[p].at[row] if n == 128
                   else idx_b[p].at[row, pl.ds(coff, n)])
            pltpu.async_copy(table_hbm.at[src],
                             emb_b[eb].at[pl.ds(doff, n)], sem_b[eb])

    def drain(half, p, eb):
        for row, coff, n, doff in _GATHERS[half]:
            src = (idx_b[p].at[row] if n == 128
                   else idx_b[p].at[row, pl.ds(coff, n)])
            pltpu.make_async_copy(table_hbm.at[src],
                                  emb_b[eb].at[pl.ds(doff, n)],
                                  sem_b[eb]).wait()

    def accum(half, eb):
        emb_v = emb_b[eb]
        for q in range(HC):
            r = half * HC + q

            def j_body(j, accs):
                e0, o0, e1, o1 = accs
                off = q * H + 4 * j
                for g, sel in ((0, 0), (32, 1)):
                    s = ((emb_v[off, pl.ds(g, 32)] +
                          emb_v[off + 1, pl.ds(g, 32)]) +
                         (emb_v[off + 2, pl.ds(g, 32)] +
                          emb_v[off + 3, pl.ds(g, 32)]))
                    a, bb = plsc.unpack(s, format=plsc.PackFormat.INTERLEAVED)
                    if sel == 0:
                        e0, o0 = e0 + a, o0 + bb
                    else:
                        e1, o1 = e1 + a, o1 + bb
                return (e0, o0, e1, o1)

            z = jnp.zeros((16,), jnp.float32)
            e0, o0, e1, o1 = lax.fori_loop(0, H // 4, j_body, (z, z, z, z),
                                           unroll=5)
            cb = 64 * (r % 2)
            out_v[r // 2, pl.ds(cb, 16)] = e0
            out_v[r // 2, pl.ds(cb + 16, 16)] = o0
            out_v[r // 2, pl.ds(cb + 32, 16)] = e1
            out_v[r // 2, pl.ds(cb + 48, 16)] = o1

    def flush_out(ci):
        pltpu.sync_copy(out_v, out_hbm.at[pl.ds((base + ci * C) // 2, HC)])

    stage_idx(0, 0)
    fire(0, 0, 0)

    def chunk_body(ci, carry):
        p = lax.rem(ci, 2)

        def do(p):
            fire(1, p, 1)
            stage_idx(ci + 1, 1 - p)
            drain(0, p, 0)
            accum(0, 0)
            fire(0, 1 - p, 0)
            drain(1, p, 1)
            accum(1, 1)
            flush_out(ci)

        lax.cond(p == 0, lambda: do(0), lambda: do(1))
        return carry

    lax.fori_loop(0, NCHUNK - 1, chunk_body, 0)
    pl_last = (NCHUNK - 1) % 2
    fire(1, pl_last, 1)
    drain(0, pl_last, 0)
    accum(0, 0)
    drain(1, pl_last, 1)
    accum(1, 1)
    flush_out(NCHUNK - 1)


@jax.jit
def _sc_sums(table_bf16, qw128):
    mesh = plsc.VectorSubcoreMesh(core_axis_name="c", subcore_axis_name="s")
    f = pl.kernel(
        _sc_body,
        mesh=mesh,
        compiler_params=pltpu.CompilerParams(
            use_tc_tiling_on_sc=False, needs_layout_passes=False),
        out_type=jax.ShapeDtypeStruct((B // 2, 2 * E), jnp.float32),
        scratch_types=[
            pltpu.VMEM((NIDX, 128), jnp.int32),
            pltpu.VMEM((NIDX, 128), jnp.int32),
            pltpu.VMEM((FLAT, E), jnp.bfloat16),
            pltpu.VMEM((FLAT, E), jnp.bfloat16),
            pltpu.VMEM((HC, 2 * E), jnp.float32),
            pltpu.SemaphoreType.DMA,
            pltpu.SemaphoreType.DMA,
        ],
    )
    return f(table_bf16, qw128)


def _proj_body(x_ref, w_ref, b_ref, o_ref):
    o_ref[...] = jnp.tanh(
        jnp.dot(x_ref[...], w_ref[...], preferred_element_type=jnp.float32)
        + b_ref[...])


@jax.jit
def _proj(sums128, w2, b2):
    blk = 2048
    return pl.pallas_call(
        _proj_body,
        grid=(B // 2 // blk,),
        in_specs=[
            pl.BlockSpec((blk, 2 * E), lambda i: (i, 0)),
            pl.BlockSpec((2 * E, 2 * E), lambda i: (0, 0)),
            pl.BlockSpec((1, 2 * E), lambda i: (0, 0)),
        ],
        out_specs=pl.BlockSpec((blk, 2 * E), lambda i: (i, 0)),
        out_shape=jax.ShapeDtypeStruct((B // 2, 2 * E), jnp.float32),
    )(sums128, w2, b2)


_PERM = np.concatenate([np.arange(0, 32, 2), np.arange(1, 32, 2),
                        np.arange(32, 64, 2), np.arange(33, 64, 2)])


def kernel(items, query_words, word_emb, q_weight, q_bias):
    qw128 = query_words.reshape(B * H // 128, 128)
    sums128 = _sc_sums(word_emb.astype(jnp.bfloat16), qw128)
    wt = (q_weight.T * (1.0 / H))[_PERM, :]
    w2 = jnp.zeros((2 * E, 2 * E), wt.dtype)
    w2 = w2.at[:E, :E].set(wt).at[E:, E:].set(wt)
    b2 = jnp.concatenate([q_bias, q_bias]).reshape(1, 2 * E)
    out128 = _proj(sums128, w2, b2)
    return out128.reshape(B, E)
